# baseline (speedup 1.0000x reference)
import functools

import jax
import jax.numpy as jnp
from jax import lax
from jax.experimental import pallas as pl
from jax.experimental.pallas import tpu as pltpu
from jax.experimental.pallas import tpu_sc as plsc

N = 10000
NPAD = 10240
E = 320000
IN_DIM = 128
HID = 256
OUT_DIM = 128
FW = 128

NC = 2
NS = 16
EB = 128
NBT = 160
EP = NS * NBT * EB
SB = 80
NH = NPAD // NC
NHT = NH // NS
RB = 512
NRB = NPAD // RB
NPT = NPAD // NS

_mesh = plsc.VectorSubcoreMesh(core_axis_name="c", subcore_axis_name="s")
_sc_params = pltpu.CompilerParams(needs_layout_passes=False)


@functools.partial(
    pl.kernel,
    out_type=jax.ShapeDtypeStruct((NC, NPAD), jnp.float32),
    mesh=_mesh,
    scratch_types=[
        pltpu.VMEM((SB, EB), jnp.int32),
        pltpu.VMEM((SB * EB,), jnp.float32),
        pltpu.VMEM((NPT,), jnp.float32),
        pltpu.VMEM_SHARED((NPAD,), jnp.float32),
        pltpu.SemaphoreType.DMA,
    ],
    compiler_params=_sc_params,
)
def _deg_kernel(dst2_hbm, ew_hbm, out_hbm, dsti_v, ew_v, zb_v, acc_sh, sem):
    c = lax.axis_index("c")
    s = lax.axis_index("s")
    zeros16 = jnp.zeros((16,), jnp.float32)

    @pl.loop(0, NPT // 16)
    def _(r):
        zb_v[pl.ds(r * 16, 16)] = zeros16

    pltpu.sync_copy(zb_v, acc_sh.at[pl.ds(s * NPT, NPT)])
    plsc.subcore_barrier()

    wid = c * NS + s
    pltpu.sync_copy(dst2_hbm.at[pl.ds(wid * SB, SB)], dsti_v)
    pltpu.sync_copy(ew_hbm.at[pl.ds(wid * SB * EB, SB * EB)], ew_v)

    for grp in range(SB // 16):

        @pl.loop(grp * 16, (grp + 1) * 16)
        def _(j):
            pltpu.async_copy(ew_v.at[pl.ds(j * EB, EB)],
                             acc_sh.at[dsti_v.at[j]], sem, add=True)

        @pl.loop(0, 16)
        def _(j):
            pltpu.make_async_copy(ew_v.at[pl.ds(0, EB)],
                                  acc_sh.at[dsti_v.at[0]], sem).wait()

    plsc.subcore_barrier()
    pltpu.sync_copy(acc_sh.at[pl.ds(s * NPT, NPT)],
                    out_hbm.at[c, pl.ds(s * NPT, NPT)])


@functools.partial(
    pl.kernel,
    out_type=jax.ShapeDtypeStruct((NPAD, FW), jnp.float32),
    mesh=_mesh,
    scratch_types=[
        pltpu.VMEM((SB * EB,), jnp.int32),
        pltpu.VMEM((SB, EB), jnp.int32),
        pltpu.VMEM((SB * EB,), jnp.int32),
        pltpu.VMEM((SB * EB,), jnp.float32),
        pltpu.VMEM((2, EB, FW), jnp.float32),
        pltpu.VMEM((64, FW), jnp.float32),
        pltpu.VMEM_SHARED((NH, FW), jnp.float32),
    ] + [pltpu.SemaphoreType.DMA] * 4,
    compiler_params=_sc_params,
)
def _agg(t_hbm, src_hbm, dst_hbm, ew_hbm, out_hbm,
         srci_v, dsti_v, dstf_v, ew_v, rows_v, zb_v, acc_sh,
         g0, g1, s0, s1):
    c = lax.axis_index("c")
    s = lax.axis_index("s")
    gsems = (g0, g1)
    ssems = (s0, s1)
    lo = c * NH
    zeros16 = jnp.zeros((16,), jnp.float32)
    iota16 = lax.iota(jnp.int32, 16)

    @pl.loop(0, 64)
    def _(r):
        for g in range(FW // 16):
            zb_v[r, pl.ds(g * 16, 16)] = zeros16

    for k in range(NHT // 64):
        pltpu.sync_copy(zb_v, acc_sh.at[pl.ds(s * NHT + k * 64, 64)])
    plsc.subcore_barrier()

    def issue_gather(j, slot):
        pltpu.async_copy(t_hbm.at[srci_v.at[pl.ds(j * EB, EB)]],
                         rows_v.at[slot], gsems[slot])

    def wait_gather(slot):
        pltpu.make_async_copy(t_hbm.at[srci_v.at[pl.ds(0, EB)]],
                              rows_v.at[slot], gsems[slot]).wait()

    def wait_scat(slot):
        pltpu.make_async_copy(rows_v.at[slot], acc_sh.at[dsti_v.at[0]],
                              ssems[slot]).wait()

    def scale_rows(slot, j):
        @pl.loop(0, EB, unroll=8)
        def _(r):
            w = plsc.load_gather(ew_v, [jnp.full((16,), j * EB + r,
                                                 jnp.int32)])
            for g in range(FW // 16):
                sl = pl.ds(g * 16, 16)
                rows_v[slot, r, sl] = rows_v[slot, r, sl] * w

    for h in range(NBT // SB):
        ebase = s * NBT * EB + h * SB * EB
        pltpu.sync_copy(src_hbm.at[pl.ds(ebase, SB * EB)], srci_v)
        pltpu.sync_copy(dst_hbm.at[pl.ds(ebase, SB * EB)], dstf_v)
        pltpu.sync_copy(ew_hbm.at[pl.ds(ebase, SB * EB)], ew_v)

        @pl.loop(0, SB)
        def _(j):
            for g in range(EB // 16):
                f = j * EB + g * 16
                fi = jnp.full((16,), f, jnp.int32) + iota16
                d = plsc.load_gather(dstf_v, [fi])
                w = plsc.load_gather(ew_v, [fi])
                own = (d >= lo) & (d < lo + NH)
                plsc.store_scatter(ew_v, [fi], jnp.where(own, w, 0.0))
                dsti_v[j, pl.ds(g * 16, 16)] = jnp.clip(d - lo, 0, NH - 1)

        issue_gather(0, 0)

        @pl.loop(0, SB // 2)
        def _(jj):
            for k in range(2):
                j = jj * 2 + k

                @pl.when(j >= 1)
                def _():
                    wait_scat((k + 1) % 2)

                @pl.when(j + 1 < SB)
                def _():
                    issue_gather(j + 1, (k + 1) % 2)

                wait_gather(k)
                scale_rows(k, j)
                pltpu.async_copy(rows_v.at[k], acc_sh.at[dsti_v.at[j]],
                                 ssems[k], add=True)

        wait_scat(1)

    plsc.subcore_barrier()
    pltpu.sync_copy(acc_sh.at[pl.ds(s * NHT, NHT)],
                    out_hbm.at[pl.ds(c * NH + s * NHT, NHT)])


def _prep_body(degp_ref, x_ref, xs_ref, dinv_ref):
    i = pl.program_id(0)
    deg = degp_ref[0, i, :] + degp_ref[1, i, :] + 1.0
    dinv = lax.rsqrt(deg)
    dinv_ref[...] = dinv[None, None, :]
    xs_ref[...] = x_ref[...] * dinv[:, None]


def _prep(degp, x):
    return pl.pallas_call(
        _prep_body,
        grid=(NRB,),
        in_specs=[
            pl.BlockSpec((NC, NRB, RB), lambda i: (0, 0, 0)),
            pl.BlockSpec((RB, IN_DIM), lambda i: (i, 0)),
        ],
        out_specs=[
            pl.BlockSpec((RB, IN_DIM), lambda i: (i, 0)),
            pl.BlockSpec((1, 1, RB), lambda i: (i, 0, 0)),
        ],
        out_shape=[
            jax.ShapeDtypeStruct((NPAD, IN_DIM), jnp.float32),
            jax.ShapeDtypeStruct((NRB, 1, RB), jnp.float32),
        ],
    )(degp, x)


def _mid_body(g1_ref, xs_ref, dinv_ref, w1_ref, b1_ref, w2_ref, h2_ref):
    dinv = dinv_ref[0, 0, :]
    t = (g1_ref[...] + xs_ref[...]) * dinv[:, None]
    t = jnp.dot(t, w1_ref[...], preferred_element_type=jnp.float32)
    t = jnp.maximum(t + b1_ref[...][None, :], 0.0)
    h2 = jnp.dot(t, w2_ref[...], preferred_element_type=jnp.float32)
    h2_ref[...] = h2 * dinv[:, None]


def _mid(g1, xs, dinv, w1, b1, w2):
    full = pl.BlockSpec((RB, IN_DIM), lambda i: (i, 0))
    return pl.pallas_call(
        _mid_body,
        grid=(NRB,),
        in_specs=[
            full, full,
            pl.BlockSpec((1, 1, RB), lambda i: (i, 0, 0)),
            pl.BlockSpec((IN_DIM, HID), lambda i: (0, 0)),
            pl.BlockSpec((HID,), lambda i: (0,)),
            pl.BlockSpec((HID, OUT_DIM), lambda i: (0, 0)),
        ],
        out_specs=pl.BlockSpec((RB, OUT_DIM), lambda i: (i, 0)),
        out_shape=jax.ShapeDtypeStruct((NPAD, OUT_DIM), jnp.float32),
    )(g1, xs, dinv, w1, b1, w2)


def _fin_body(g2_ref, h2_ref, dinv_ref, b2_ref, out_ref):
    dinv = dinv_ref[0, 0, :]
    o = g2_ref[...] + h2_ref[...]
    out_ref[...] = o * dinv[:, None] + b2_ref[...][None, :]


def _fin(g2, h2, dinv, b2):
    full = pl.BlockSpec((RB, OUT_DIM), lambda i: (i, 0))
    return pl.pallas_call(
        _fin_body,
        grid=(NRB,),
        in_specs=[
            full, full,
            pl.BlockSpec((1, 1, RB), lambda i: (i, 0, 0)),
            pl.BlockSpec((OUT_DIM,), lambda i: (0,)),
        ],
        out_specs=full,
        out_shape=jax.ShapeDtypeStruct((NPAD, OUT_DIM), jnp.float32),
    )(g2, h2, dinv, b2)


def kernel(node_features, edge_index, edge_weight, W1, b1, W2, b2):
    pad = EP - E
    src = jnp.pad(edge_index[0], (0, pad))
    dst = jnp.pad(edge_index[1], (0, pad))
    ew = jnp.pad(edge_weight, (0, pad))
    dst2 = dst.reshape(EP // EB, EB)
    xpad = jnp.pad(node_features, ((0, NPAD - N), (0, 0)))

    degp = _deg_kernel(dst2, ew)
    degp = degp.reshape(NC, NRB, RB)
    xs, dinv = _prep(degp, xpad)
    g1 = _agg(xs, src, dst, ew)
    h2 = _mid(g1, xs, dinv, W1, b1, W2)
    g2 = _agg(h2, src, dst, ew)
    return _fin(g2, h2, dinv, b2)[:N]


# speedup vs baseline: 1.0432x; 1.0432x over previous
import functools

import jax
import jax.numpy as jnp
from jax import lax
from jax.experimental import pallas as pl
from jax.experimental.pallas import tpu as pltpu
from jax.experimental.pallas import tpu_sc as plsc

N = 10000
NPAD = 10240
E = 320000
IN_DIM = 128
HID = 256
OUT_DIM = 128
FW = 128

NC = 2
NS = 16
EB = 128
NBT = 160
EP = NS * NBT * EB
SB = 80
NH = NPAD // NC
NHT = NH // NS
RB = 512
NRB = NPAD // RB
NPT = NPAD // NS

_mesh = plsc.VectorSubcoreMesh(core_axis_name="c", subcore_axis_name="s")
_sc_params = pltpu.CompilerParams(needs_layout_passes=False)


@functools.partial(
    pl.kernel,
    out_type=jax.ShapeDtypeStruct((NC, NPAD), jnp.float32),
    mesh=_mesh,
    scratch_types=[
        pltpu.VMEM((SB, EB), jnp.int32),
        pltpu.VMEM((SB * EB,), jnp.float32),
        pltpu.VMEM((NPT,), jnp.float32),
        pltpu.VMEM_SHARED((NPAD,), jnp.float32),
        pltpu.SemaphoreType.DMA,
    ],
    compiler_params=_sc_params,
)
def _deg_kernel(dst2_hbm, ew_hbm, out_hbm, dsti_v, ew_v, zb_v, acc_sh, sem):
    c = lax.axis_index("c")
    s = lax.axis_index("s")
    zeros16 = jnp.zeros((16,), jnp.float32)

    @pl.loop(0, NPT // 16)
    def _(r):
        zb_v[pl.ds(r * 16, 16)] = zeros16

    pltpu.sync_copy(zb_v, acc_sh.at[pl.ds(s * NPT, NPT)])
    plsc.subcore_barrier()

    wid = c * NS + s
    pltpu.sync_copy(dst2_hbm.at[pl.ds(wid * SB, SB)], dsti_v)
    pltpu.sync_copy(ew_hbm.at[pl.ds(wid * SB * EB, SB * EB)], ew_v)

    for grp in range(SB // 16):

        @pl.loop(grp * 16, (grp + 1) * 16)
        def _(j):
            pltpu.async_copy(ew_v.at[pl.ds(j * EB, EB)],
                             acc_sh.at[dsti_v.at[j]], sem, add=True)

        @pl.loop(0, 16)
        def _(j):
            pltpu.make_async_copy(ew_v.at[pl.ds(0, EB)],
                                  acc_sh.at[dsti_v.at[0]], sem).wait()

    plsc.subcore_barrier()
    pltpu.sync_copy(acc_sh.at[pl.ds(s * NPT, NPT)],
                    out_hbm.at[c, pl.ds(s * NPT, NPT)])


@functools.partial(
    pl.kernel,
    out_type=jax.ShapeDtypeStruct((NPAD, FW), jnp.float32),
    mesh=_mesh,
    scratch_types=[
        pltpu.VMEM((SB * EB,), jnp.int32),
        pltpu.VMEM((SB, EB), jnp.int32),
        pltpu.VMEM((SB * EB,), jnp.int32),
        pltpu.VMEM((SB * EB,), jnp.float32),
        pltpu.VMEM((2, EB, FW), jnp.float32),
        pltpu.VMEM((64, FW), jnp.float32),
        pltpu.VMEM_SHARED((NH, FW), jnp.float32),
    ] + [pltpu.SemaphoreType.DMA] * 4,
    compiler_params=_sc_params,
)
def _agg(t_hbm, src_hbm, dst_hbm, ew_hbm, out_hbm,
         srci_v, dsti_v, dstf_v, ew_v, rows_v, zb_v, acc_sh,
         g0, g1, s0, s1):
    c = lax.axis_index("c")
    s = lax.axis_index("s")
    gsems = (g0, g1)
    ssems = (s0, s1)
    lo = c * NH
    zeros16 = jnp.zeros((16,), jnp.float32)
    iota16 = lax.iota(jnp.int32, 16)

    @pl.loop(0, 64)
    def _(r):
        for g in range(FW // 16):
            zb_v[r, pl.ds(g * 16, 16)] = zeros16

    for k in range(NHT // 64):
        pltpu.sync_copy(zb_v, acc_sh.at[pl.ds(s * NHT + k * 64, 64)])
    plsc.subcore_barrier()

    def issue_gather(j, slot):
        pltpu.async_copy(t_hbm.at[srci_v.at[pl.ds(j * EB, EB)]],
                         rows_v.at[slot], gsems[slot])

    def wait_gather(slot):
        pltpu.make_async_copy(t_hbm.at[srci_v.at[pl.ds(0, EB)]],
                              rows_v.at[slot], gsems[slot]).wait()

    def wait_scat(slot):
        pltpu.make_async_copy(rows_v.at[slot], acc_sh.at[dsti_v.at[0]],
                              ssems[slot]).wait()

    def scale_rows(slot, j):
        @plsc.parallel_loop(0, EB, unroll=8)
        def _(r):
            w = plsc.load_gather(ew_v, [jnp.full((16,), j * EB + r,
                                                 jnp.int32)])
            for g in range(FW // 16):
                sl = pl.ds(g * 16, 16)
                rows_v[slot, r, sl] = rows_v[slot, r, sl] * w

    for h in range(NBT // SB):
        ebase = s * NBT * EB + h * SB * EB
        pltpu.sync_copy(src_hbm.at[pl.ds(ebase, SB * EB)], srci_v)
        pltpu.sync_copy(dst_hbm.at[pl.ds(ebase, SB * EB)], dstf_v)
        pltpu.sync_copy(ew_hbm.at[pl.ds(ebase, SB * EB)], ew_v)

        @plsc.parallel_loop(0, SB, unroll=2)
        def _(j):
            for g in range(EB // 16):
                f = j * EB + g * 16
                fi = jnp.full((16,), f, jnp.int32) + iota16
                d = plsc.load_gather(dstf_v, [fi])
                w = plsc.load_gather(ew_v, [fi])
                own = (d >= lo) & (d < lo + NH)
                plsc.store_scatter(ew_v, [fi], jnp.where(own, w, 0.0))
                dsti_v[j, pl.ds(g * 16, 16)] = jnp.clip(d - lo, 0, NH - 1)

        issue_gather(0, 0)

        @pl.loop(0, SB // 2)
        def _(jj):
            for k in range(2):
                j = jj * 2 + k

                @pl.when(j >= 1)
                def _():
                    wait_scat((k + 1) % 2)

                @pl.when(j + 1 < SB)
                def _():
                    issue_gather(j + 1, (k + 1) % 2)

                wait_gather(k)
                scale_rows(k, j)
                pltpu.async_copy(rows_v.at[k], acc_sh.at[dsti_v.at[j]],
                                 ssems[k], add=True)

        wait_scat(1)

    plsc.subcore_barrier()
    pltpu.sync_copy(acc_sh.at[pl.ds(s * NHT, NHT)],
                    out_hbm.at[pl.ds(c * NH + s * NHT, NHT)])


def _prep_body(degp_ref, x_ref, xs_ref, dinv_ref):
    i = pl.program_id(0)
    deg = degp_ref[0, i, :] + degp_ref[1, i, :] + 1.0
    dinv = lax.rsqrt(deg)
    dinv_ref[...] = dinv[None, None, :]
    xs_ref[...] = x_ref[...] * dinv[:, None]


def _prep(degp, x):
    return pl.pallas_call(
        _prep_body,
        grid=(NRB,),
        in_specs=[
            pl.BlockSpec((NC, NRB, RB), lambda i: (0, 0, 0)),
            pl.BlockSpec((RB, IN_DIM), lambda i: (i, 0)),
        ],
        out_specs=[
            pl.BlockSpec((RB, IN_DIM), lambda i: (i, 0)),
            pl.BlockSpec((1, 1, RB), lambda i: (i, 0, 0)),
        ],
        out_shape=[
            jax.ShapeDtypeStruct((NPAD, IN_DIM), jnp.float32),
            jax.ShapeDtypeStruct((NRB, 1, RB), jnp.float32),
        ],
    )(degp, x)


def _mid_body(g1_ref, xs_ref, dinv_ref, w1_ref, b1_ref, w2_ref, h2_ref):
    dinv = dinv_ref[0, 0, :]
    t = (g1_ref[...] + xs_ref[...]) * dinv[:, None]
    t = jnp.dot(t, w1_ref[...], preferred_element_type=jnp.float32)
    t = jnp.maximum(t + b1_ref[...][None, :], 0.0)
    h2 = jnp.dot(t, w2_ref[...], preferred_element_type=jnp.float32)
    h2_ref[...] = h2 * dinv[:, None]


def _mid(g1, xs, dinv, w1, b1, w2):
    full = pl.BlockSpec((RB, IN_DIM), lambda i: (i, 0))
    return pl.pallas_call(
        _mid_body,
        grid=(NRB,),
        in_specs=[
            full, full,
            pl.BlockSpec((1, 1, RB), lambda i: (i, 0, 0)),
            pl.BlockSpec((IN_DIM, HID), lambda i: (0, 0)),
            pl.BlockSpec((HID,), lambda i: (0,)),
            pl.BlockSpec((HID, OUT_DIM), lambda i: (0, 0)),
        ],
        out_specs=pl.BlockSpec((RB, OUT_DIM), lambda i: (i, 0)),
        out_shape=jax.ShapeDtypeStruct((NPAD, OUT_DIM), jnp.float32),
    )(g1, xs, dinv, w1, b1, w2)


def _fin_body(g2_ref, h2_ref, dinv_ref, b2_ref, out_ref):
    dinv = dinv_ref[0, 0, :]
    o = g2_ref[...] + h2_ref[...]
    out_ref[...] = o * dinv[:, None] + b2_ref[...][None, :]


def _fin(g2, h2, dinv, b2):
    full = pl.BlockSpec((RB, OUT_DIM), lambda i: (i, 0))
    return pl.pallas_call(
        _fin_body,
        grid=(NRB,),
        in_specs=[
            full, full,
            pl.BlockSpec((1, 1, RB), lambda i: (i, 0, 0)),
            pl.BlockSpec((OUT_DIM,), lambda i: (0,)),
        ],
        out_specs=full,
        out_shape=jax.ShapeDtypeStruct((NPAD, OUT_DIM), jnp.float32),
    )(g2, h2, dinv, b2)


def kernel(node_features, edge_index, edge_weight, W1, b1, W2, b2):
    pad = EP - E
    src = jnp.pad(edge_index[0], (0, pad))
    dst = jnp.pad(edge_index[1], (0, pad))
    ew = jnp.pad(edge_weight, (0, pad))
    dst2 = dst.reshape(EP // EB, EB)
    xpad = jnp.pad(node_features, ((0, NPAD - N), (0, 0)))

    degp = _deg_kernel(dst2, ew)
    degp = degp.reshape(NC, NRB, RB)
    xs, dinv = _prep(degp, xpad)
    g1 = _agg(xs, src, dst, ew)
    h2 = _mid(g1, xs, dinv, W1, b1, W2)
    g2 = _agg(h2, src, dst, ew)
    return _fin(g2, h2, dinv, b2)[:N]


# speedup vs baseline: 1.7260x; 1.6545x over previous
import functools

import jax
import jax.numpy as jnp
from jax import lax
from jax.experimental import pallas as pl
from jax.experimental.pallas import tpu as pltpu
from jax.experimental.pallas import tpu_sc as plsc

N = 10000
NPAD = 10240
E = 320000
IN_DIM = 128
HID = 256
OUT_DIM = 128
FW = 128

NC = 2
NS = 16
EB = 128
EP = 327680
RB = 512
NRB = NPAD // RB
NPT = NPAD // NS
NPC = NPT // 5

_mesh = plsc.VectorSubcoreMesh(core_axis_name="c", subcore_axis_name="s")
_sc_params = pltpu.CompilerParams(needs_layout_passes=False)


@functools.partial(
    pl.kernel,
    out_type=jax.ShapeDtypeStruct((NC, NPAD), jnp.float32),
    mesh=_mesh,
    scratch_types=[
        pltpu.VMEM((80, EB), jnp.int32),
        pltpu.VMEM((80 * EB,), jnp.float32),
        pltpu.VMEM((NPT,), jnp.float32),
        pltpu.VMEM_SHARED((NPAD,), jnp.float32),
        pltpu.SemaphoreType.DMA,
    ],
    compiler_params=_sc_params,
)
def _deg_kernel(dst2_hbm, ew_hbm, out_hbm, dsti_v, ew_v, zb_v, acc_sh, sem):
    c = lax.axis_index("c")
    s = lax.axis_index("s")
    zeros16 = jnp.zeros((16,), jnp.float32)

    @pl.loop(0, NPT // 16)
    def _(r):
        zb_v[pl.ds(r * 16, 16)] = zeros16

    pltpu.sync_copy(zb_v, acc_sh.at[pl.ds(s * NPT, NPT)])
    plsc.subcore_barrier()

    wid = c * NS + s
    pltpu.sync_copy(dst2_hbm.at[pl.ds(wid * 80, 80)], dsti_v)
    pltpu.sync_copy(ew_hbm.at[pl.ds(wid * 80 * EB, 80 * EB)], ew_v)

    for grp in range(80 // 16):

        @pl.loop(grp * 16, (grp + 1) * 16)
        def _(j):
            pltpu.async_copy(ew_v.at[pl.ds(j * EB, EB)],
                             acc_sh.at[dsti_v.at[j]], sem, add=True)

        @pl.loop(0, 16)
        def _(j):
            pltpu.make_async_copy(ew_v.at[pl.ds(0, EB)],
                                  acc_sh.at[dsti_v.at[0]], sem).wait()

    plsc.subcore_barrier()
    pltpu.sync_copy(acc_sh.at[pl.ds(s * NPT, NPT)],
                    out_hbm.at[c, pl.ds(s * NPT, NPT)])


NBW = EP // (NC * NS * EB)
SB = 16

@functools.partial(
    pl.kernel,
    out_type=(
        jax.ShapeDtypeStruct((NPAD, FW), jnp.float32),
        jax.ShapeDtypeStruct((NPAD, FW), jnp.float32),
    ),
    mesh=_mesh,
    scratch_types=[
        pltpu.VMEM((SB * EB,), jnp.int32),
        pltpu.VMEM((SB, EB), jnp.int32),
        pltpu.VMEM((SB * EB,), jnp.float32),
        pltpu.VMEM((2, EB, FW), jnp.float32),
        pltpu.VMEM_SHARED((NPAD, FW), jnp.float32),
    ] + [pltpu.SemaphoreType.DMA] * 4,
    compiler_params=_sc_params,
)
def _agg(t_hbm, src_hbm, dst2_hbm, ew_hbm, o0_hbm, o1_hbm,
         srci_v, dsti_v, ew_v, rows_v, acc_sh, g0, g1, s0, s1):
    c = lax.axis_index("c")
    s = lax.axis_index("s")
    gsems = (g0, g1)
    ssems = (s0, s1)
    zeros16 = jnp.zeros((16,), jnp.float32)

    @pl.loop(0, EB)
    def _(r):
        for g in range(FW // 16):
            rows_v[0, r, pl.ds(g * 16, 16)] = zeros16

    for k in range(NPT // EB):
        pltpu.sync_copy(rows_v.at[0], acc_sh.at[pl.ds(s * NPT + k * EB, EB)])
    plsc.subcore_barrier()

    def issue_gather(j, slot):
        pltpu.async_copy(t_hbm.at[srci_v.at[pl.ds(j * EB, EB)]],
                         rows_v.at[slot], gsems[slot])

    def wait_gather(slot):
        pltpu.make_async_copy(t_hbm.at[srci_v.at[pl.ds(0, EB)]],
                              rows_v.at[slot], gsems[slot]).wait()

    def wait_scat(slot):
        pltpu.make_async_copy(rows_v.at[slot], acc_sh.at[dsti_v.at[0]],
                              ssems[slot]).wait()

    def scale_rows(slot, j):
        @plsc.parallel_loop(0, EB, unroll=8)
        def _(r):
            w = plsc.load_gather(ew_v, [jnp.full((16,), j * EB + r,
                                                 jnp.int32)])
            for g in range(FW // 16):
                sl = pl.ds(g * 16, 16)
                rows_v[slot, r, sl] = rows_v[slot, r, sl] * w

    wid = c * NS + s
    for st in range(NBW // SB):
        ebase = wid * NBW * EB + st * SB * EB
        rbase = wid * NBW + st * SB
        pltpu.sync_copy(src_hbm.at[pl.ds(ebase, SB * EB)], srci_v)
        pltpu.sync_copy(ew_hbm.at[pl.ds(ebase, SB * EB)], ew_v)
        pltpu.sync_copy(dst2_hbm.at[pl.ds(rbase, SB)], dsti_v)

        issue_gather(0, 0)

        @pl.loop(0, SB // 2)
        def _(jj):
            for k in range(2):
                j = jj * 2 + k

                @pl.when(j >= 1)
                def _():
                    wait_scat((k + 1) % 2)

                @pl.when(j + 1 < SB)
                def _():
                    issue_gather(j + 1, (k + 1) % 2)

                wait_gather(k)
                scale_rows(k, j)
                pltpu.async_copy(rows_v.at[k], acc_sh.at[dsti_v.at[j]],
                                 ssems[k], add=True)

        wait_scat(1)

    plsc.subcore_barrier()
    for k in range(NPT // NPC):
        sl = pl.ds(s * NPT + k * NPC, NPC)

        @pl.when(c == 0)
        def _():
            pltpu.sync_copy(acc_sh.at[sl], o0_hbm.at[sl])

        @pl.when(c == 1)
        def _():
            pltpu.sync_copy(acc_sh.at[sl], o1_hbm.at[sl])


def _prep_body(degp_ref, x_ref, xs_ref, dinv_ref):
    i = pl.program_id(0)
    deg = degp_ref[0, i, :] + degp_ref[1, i, :] + 1.0
    dinv = lax.rsqrt(deg)
    dinv_ref[...] = dinv[None, None, :]
    xs_ref[...] = x_ref[...] * dinv[:, None]


def _prep(degp, x):
    return pl.pallas_call(
        _prep_body,
        grid=(NRB,),
        in_specs=[
            pl.BlockSpec((NC, NRB, RB), lambda i: (0, 0, 0)),
            pl.BlockSpec((RB, IN_DIM), lambda i: (i, 0)),
        ],
        out_specs=[
            pl.BlockSpec((RB, IN_DIM), lambda i: (i, 0)),
            pl.BlockSpec((1, 1, RB), lambda i: (i, 0, 0)),
        ],
        out_shape=[
            jax.ShapeDtypeStruct((NPAD, IN_DIM), jnp.float32),
            jax.ShapeDtypeStruct((NRB, 1, RB), jnp.float32),
        ],
    )(degp, x)


def _mid_body(p0_ref, p1_ref, xs_ref, dinv_ref, w1_ref, b1_ref, w2_ref,
              h2_ref):
    dinv = dinv_ref[0, 0, :]
    t = (p0_ref[...] + p1_ref[...] + xs_ref[...]) * dinv[:, None]
    t = jnp.dot(t, w1_ref[...], preferred_element_type=jnp.float32)
    t = jnp.maximum(t + b1_ref[...][None, :], 0.0)
    h2 = jnp.dot(t, w2_ref[...], preferred_element_type=jnp.float32)
    h2_ref[...] = h2 * dinv[:, None]


def _mid(p0, p1, xs, dinv, w1, b1, w2):
    full = pl.BlockSpec((RB, IN_DIM), lambda i: (i, 0))
    return pl.pallas_call(
        _mid_body,
        grid=(NRB,),
        in_specs=[
            full, full, full,
            pl.BlockSpec((1, 1, RB), lambda i: (i, 0, 0)),
            pl.BlockSpec((IN_DIM, HID), lambda i: (0, 0)),
            pl.BlockSpec((HID,), lambda i: (0,)),
            pl.BlockSpec((HID, OUT_DIM), lambda i: (0, 0)),
        ],
        out_specs=pl.BlockSpec((RB, OUT_DIM), lambda i: (i, 0)),
        out_shape=jax.ShapeDtypeStruct((NPAD, OUT_DIM), jnp.float32),
    )(p0, p1, xs, dinv, w1, b1, w2)


def _fin_body(p0_ref, p1_ref, h2_ref, dinv_ref, b2_ref, out_ref):
    dinv = dinv_ref[0, 0, :]
    o = p0_ref[...] + p1_ref[...] + h2_ref[...]
    out_ref[...] = o * dinv[:, None] + b2_ref[...][None, :]


def _fin(p0, p1, h2, dinv, b2):
    full = pl.BlockSpec((RB, OUT_DIM), lambda i: (i, 0))
    return pl.pallas_call(
        _fin_body,
        grid=(NRB,),
        in_specs=[
            full, full, full,
            pl.BlockSpec((1, 1, RB), lambda i: (i, 0, 0)),
            pl.BlockSpec((OUT_DIM,), lambda i: (0,)),
        ],
        out_specs=full,
        out_shape=jax.ShapeDtypeStruct((NPAD, OUT_DIM), jnp.float32),
    )(p0, p1, h2, dinv, b2)


def kernel(node_features, edge_index, edge_weight, W1, b1, W2, b2):
    pad = EP - E
    src = jnp.pad(edge_index[0], (0, pad))
    dst = jnp.pad(edge_index[1], (0, pad))
    ew = jnp.pad(edge_weight, (0, pad))
    dst2 = dst.reshape(EP // EB, EB)
    xpad = jnp.pad(node_features, ((0, NPAD - N), (0, 0)))

    degp = _deg_kernel(dst2, ew)
    degp = degp.reshape(NC, NRB, RB)
    xs, dinv = _prep(degp, xpad)
    q0, q1 = _agg(xs, src, dst2, ew)
    h2 = _mid(q0, q1, xs, dinv, W1, b1, W2)
    p0, p1 = _agg(h2, src, dst2, ew)
    return _fin(p0, p1, h2, dinv, b2)[:N]
